# baseline (speedup 1.0000x reference)
import functools

import jax
import jax.numpy as jnp
from jax import lax
from jax.experimental import pallas as pl
from jax.experimental.pallas import tpu as pltpu
from jax.experimental.pallas import tpu_sc as plsc

NW = 32
BW = 128
L = 16


def _make_gather(S, B0, V, D):
  mesh = plsc.VectorSubcoreMesh(core_axis_name="c", subcore_axis_name="s")
  assert S % 2 == 0 and B0 == NW * BW

  @functools.partial(
      pl.kernel,
      mesh=mesh,
      out_type=jax.ShapeDtypeStruct((S, D, B0), jnp.float32),
      compiler_params=pltpu.CompilerParams(needs_layout_passes=False),
      scratch_types=[
          pltpu.VMEM((S, BW), jnp.int32),
          pltpu.VMEM((2, BW, 2 * D), jnp.float32),
          pltpu.VMEM((2, D, BW), jnp.float32),
          pltpu.SemaphoreType.DMA,
          pltpu.SemaphoreType.DMA,
          pltpu.SemaphoreType.DMA,
          pltpu.SemaphoreType.DMA,
      ],
  )
  def body(idx_hbm, table_hbm, out_hbm, idx_v, buf_a, buf_b, g0, g1, o0, o1):
    wid = lax.axis_index("s") * 2 + lax.axis_index("c")
    b0 = wid * BW
    gsem = (g0, g1)
    osem = (o0, o1)

    pltpu.sync_copy(idx_hbm.at[:, pl.ds(b0, BW)], idx_v)

    def gather(s, p):
      return pltpu.make_async_copy(
          table_hbm.at[idx_v.at[s]], buf_a.at[p], gsem[p])

    def store(s, p):
      return pltpu.make_async_copy(
          buf_b.at[p], out_hbm.at[s].at[:, pl.ds(b0, BW)], osem[p])

    lane = lax.iota(jnp.int32, L)
    row_ids = [lane + g * L for g in range(BW // L)]

    def transpose(p):
      src = buf_a.at[p]
      dst = buf_b.at[p]

      def step(dd, carry):
        for q in range(2):
          diag = (lane + dd * 2 + q) & (D - 1)
          for g in range(BW // L):
            vals = plsc.load_gather(src, [row_ids[g], diag])
            plsc.store_scatter(dst, [diag, row_ids[g]], vals)
        return carry

      lax.fori_loop(0, D // 2, step, 0)

    gather(0, 0).start()

    def half(s, p):
      np_ = 1 - p

      @pl.when(s + 1 < S)
      def _():
        gather(s + 1, np_).start()

      gather(s, p).wait()

      @pl.when(s >= 2)
      def _():
        store(s - 2, p).wait()

      transpose(p)
      store(s, p).start()

    def pair(k, carry):
      half(2 * k, 0)
      half(2 * k + 1, 1)
      return carry

    lax.fori_loop(0, S // 2, pair, 0)
    store(S - 2, 0).wait()
    store(S - 1, 1).wait()

  return body


def _make_pad_transpose(V, D, W=512):
  grid = (V + W - 1) // W

  @functools.partial(
      pl.pallas_call,
      grid=(grid,),
      in_specs=[pl.BlockSpec((D, W), lambda i: (0, i))],
      out_specs=pl.BlockSpec((W, 2 * D), lambda i: (i, 0)),
      out_shape=jax.ShapeDtypeStruct((V, 2 * D), jnp.float32),
  )
  def body(x_ref, o_ref):
    x = x_ref[...]
    xt = lax.dot_general(
        x, jnp.eye(D, dtype=jnp.float32),
        (((0,), (0,)), ((), ())),
        preferred_element_type=jnp.float32,
    )
    o_ref[...] = jnp.concatenate(
        [xt, jnp.zeros((W, D), jnp.float32)], axis=1)

  return body


def kernel(inputs, weight):
  B0, S = inputs.shape
  V, D = weight.shape
  idx_t = inputs.T.astype(jnp.int32)
  table = _make_pad_transpose(V, D)(weight.T)
  out_t = _make_gather(S, B0, V, D)(idx_t, table)
  return out_t.transpose(2, 0, 1)


# speedup vs baseline: 1.8932x; 1.8932x over previous
import functools

import jax
import jax.numpy as jnp
from jax import lax
from jax.experimental import pallas as pl
from jax.experimental.pallas import tpu as pltpu
from jax.experimental.pallas import tpu_sc as plsc

NW = 32
BW = 128
L = 16


def _make_gather(S, B0, V, D):
  mesh = plsc.VectorSubcoreMesh(core_axis_name="c", subcore_axis_name="s")
  assert S % 2 == 0 and B0 == NW * BW

  @functools.partial(
      pl.kernel,
      mesh=mesh,
      out_type=jax.ShapeDtypeStruct((S, D, B0), jnp.float32),
      compiler_params=pltpu.CompilerParams(needs_layout_passes=False),
      scratch_types=[
          pltpu.VMEM((S, BW), jnp.int32),
          pltpu.VMEM((2, BW, 2 * D), jnp.float32),
          pltpu.VMEM((2, D, BW), jnp.float32),
          pltpu.SemaphoreType.DMA,
          pltpu.SemaphoreType.DMA,
          pltpu.SemaphoreType.DMA,
          pltpu.SemaphoreType.DMA,
      ],
  )
  def body(idx_hbm, table_hbm, out_hbm, idx_v, buf_a, buf_b, g0, g1, o0, o1):
    wid = lax.axis_index("s") * 2 + lax.axis_index("c")
    b0 = wid * BW
    gsem = (g0, g1)
    osem = (o0, o1)

    pltpu.sync_copy(idx_hbm.at[:, pl.ds(b0, BW)], idx_v)

    def gather(s, p):
      return pltpu.make_async_copy(
          table_hbm.at[idx_v.at[s]], buf_a.at[p], gsem[p])

    def store(s, p):
      return pltpu.make_async_copy(
          buf_b.at[p], out_hbm.at[s].at[:, pl.ds(b0, BW)], osem[p])

    lane = lax.iota(jnp.int32, L)
    row_ids = [lane + g * L for g in range(BW // L)]

    def transpose(p):
      src = buf_a.at[p]
      dst = buf_b.at[p]

      def step(dd, carry):
        for q in range(2):
          diag = (lane + dd * 2 + q) & (D - 1)
          for g in range(BW // L):
            vals = plsc.load_gather(src, [row_ids[g], diag])
            plsc.store_scatter(dst, [diag, row_ids[g]], vals)
        return carry

      lax.fori_loop(0, D // 2, step, 0)

    gather(0, 0).start()

    def half(s, p):
      np_ = 1 - p

      @pl.when(s + 1 < S)
      def _():
        gather(s + 1, np_).start()

      gather(s, p).wait()

      @pl.when(s >= 2)
      def _():
        store(s - 2, p).wait()

      transpose(p)
      store(s, p).start()

    def pair(k, carry):
      half(2 * k, 0)
      half(2 * k + 1, 1)
      return carry

    lax.fori_loop(0, S // 2, pair, 0)
    store(S - 2, 0).wait()
    store(S - 1, 1).wait()

  return body


def _make_pad_transpose(V, D, W=4096):
  grid = (V + W - 1) // W

  @functools.partial(
      pl.pallas_call,
      grid=(grid,),
      in_specs=[pl.BlockSpec((D, W), lambda i: (0, i))],
      out_specs=pl.BlockSpec((W, 2 * D), lambda i: (i, 0)),
      out_shape=jax.ShapeDtypeStruct((V, 2 * D), jnp.float32),
  )
  def body(x_ref, o_ref):
    x = x_ref[...]
    xt = lax.dot_general(
        x, jnp.eye(D, dtype=jnp.float32),
        (((0,), (0,)), ((), ())),
        preferred_element_type=jnp.float32,
        precision=lax.Precision.HIGHEST,
    )
    o_ref[...] = jnp.concatenate(
        [xt, jnp.zeros((W, D), jnp.float32)], axis=1)

  return body


def kernel(inputs, weight):
  B0, S = inputs.shape
  V, D = weight.shape
  idx_t = inputs.T.astype(jnp.int32)
  table = _make_pad_transpose(V, D)(weight.T)
  out_t = _make_gather(S, B0, V, D)(idx_t, table)
  return out_t.transpose(2, 0, 1)


# speedup vs baseline: 1.9948x; 1.0537x over previous
import functools

import jax
import jax.numpy as jnp
from jax import lax
from jax.experimental import pallas as pl
from jax.experimental.pallas import tpu as pltpu
from jax.experimental.pallas import tpu_sc as plsc

NW = 32
BW = 128
L = 16


def _make_gather(S, B0, V, D):
  mesh = plsc.VectorSubcoreMesh(core_axis_name="c", subcore_axis_name="s")
  assert S % 2 == 0 and B0 == NW * BW

  @functools.partial(
      pl.kernel,
      mesh=mesh,
      out_type=jax.ShapeDtypeStruct((S, D, B0), jnp.float32),
      compiler_params=pltpu.CompilerParams(needs_layout_passes=False),
      scratch_types=[
          pltpu.VMEM((S, BW), jnp.int32),
          pltpu.VMEM((2, BW, 2 * D), jnp.float32),
          pltpu.VMEM((2, D, BW), jnp.float32),
          pltpu.SemaphoreType.DMA,
          pltpu.SemaphoreType.DMA,
          pltpu.SemaphoreType.DMA,
          pltpu.SemaphoreType.DMA,
      ],
  )
  def body(idx_hbm, table_hbm, out_hbm, idx_v, buf_a, buf_b, g0, g1, o0, o1):
    wid = lax.axis_index("s") * 2 + lax.axis_index("c")
    b0 = wid * BW
    gsem = (g0, g1)
    osem = (o0, o1)

    pltpu.sync_copy(idx_hbm.at[:, pl.ds(b0, BW)], idx_v)

    def gather(s, p):
      return pltpu.make_async_copy(
          table_hbm.at[idx_v.at[s]], buf_a.at[p], gsem[p])

    def store(s, p):
      return pltpu.make_async_copy(
          buf_b.at[p], out_hbm.at[s].at[:, pl.ds(b0, BW)], osem[p])

    lane = lax.iota(jnp.int32, L)
    row_ids = [lane + g * L for g in range(BW // L)]

    def transpose(p):
      src = buf_a.at[p]
      dst = buf_b.at[p]

      def step(dd, carry):
        for q in range(4):
          diag = (lane + dd * 4 + q) & (D - 1)
          for g in range(BW // L):
            vals = plsc.load_gather(src, [row_ids[g], diag])
            plsc.store_scatter(dst, [diag, row_ids[g]], vals)
        return carry

      lax.fori_loop(0, D // 4, step, 0)

    gather(0, 0).start()

    def half(s, p):
      np_ = 1 - p

      @pl.when(s + 1 < S)
      def _():
        gather(s + 1, np_).start()

      gather(s, p).wait()

      @pl.when(s >= 2)
      def _():
        store(s - 2, p).wait()

      transpose(p)
      store(s, p).start()

    def pair(k, carry):
      half(2 * k, 0)
      half(2 * k + 1, 1)
      return carry

    lax.fori_loop(0, S // 2, pair, 0)
    store(S - 2, 0).wait()
    store(S - 1, 1).wait()

  return body


def _make_pad_transpose(V, D, W=8192):
  grid = (V + W - 1) // W

  @functools.partial(
      pl.pallas_call,
      grid=(grid,),
      in_specs=[pl.BlockSpec((D, W), lambda i: (0, i))],
      out_specs=pl.BlockSpec((W, 2 * D), lambda i: (i, 0)),
      out_shape=jax.ShapeDtypeStruct((V, 2 * D), jnp.float32),
  )
  def body(x_ref, o_ref):
    x = x_ref[...]
    xt = lax.dot_general(
        x, jnp.eye(D, dtype=jnp.float32),
        (((0,), (0,)), ((), ())),
        preferred_element_type=jnp.float32,
        precision=lax.Precision.HIGHEST,
    )
    o_ref[...] = jnp.concatenate(
        [xt, jnp.zeros((W, D), jnp.float32)], axis=1)

  return body


def kernel(inputs, weight):
  B0, S = inputs.shape
  V, D = weight.shape
  idx_t = inputs.T.astype(jnp.int32)
  table = _make_pad_transpose(V, D)(weight.T)
  out_t = _make_gather(S, B0, V, D)(idx_t, table)
  return out_t.transpose(2, 0, 1)
